# Initial kernel scaffold; baseline (speedup 1.0000x reference)
import jax
import jax.numpy as jnp
from jax.experimental import pallas as pl


def kernel(cncpt_ids, edge_index, concept_table, W1, b1, W2, b2):
    raise NotImplementedError("write your pallas kernel here")


# speedup vs baseline: 8.4306x; 8.4306x over previous
import functools

import jax
import jax.numpy as jnp
from jax import lax
from jax.experimental import pallas as pl
from jax.experimental.pallas import tpu as pltpu
from jax.experimental.pallas import tpu_sc as plsc

N_NODES = 10000
N_EDGES = 320000
CONCEPT_NUM = 100000
D = 128

NC = 2
NS = 16
NW = NC * NS

N_PAD = 10240
EMB_CHUNKS = 5
EMB_CHUNK = 64
ROWS_PER_W = N_PAD // NW

E_PER_W = N_EDGES // NW
E_CHUNKS = 80
E_CHUNK = 125

N_AGG = 10240
ROWS_PER_TILE = N_AGG // NS

_mesh = plsc.VectorSubcoreMesh(core_axis_name="c", subcore_axis_name="s")


@functools.partial(
    pl.kernel,
    out_type=jax.ShapeDtypeStruct((N_PAD, D), jnp.float32),
    mesh=_mesh,
    scratch_types=[
        pltpu.VMEM((EMB_CHUNKS, EMB_CHUNK), jnp.int32),
        pltpu.VMEM((EMB_CHUNK, D), jnp.float32),
        pltpu.SemaphoreType.DMA,
    ],
)
def _embed_gather(table_hbm, cid_hbm, feats_hbm, idx_v, buf, sem):
    c = lax.axis_index("c")
    s = lax.axis_index("s")
    w = c * NS + s
    pltpu.sync_copy(cid_hbm.at[w], idx_v)
    for j in range(EMB_CHUNKS):
        pltpu.async_copy(table_hbm.at[idx_v.at[j]], buf, sem).wait()
        pltpu.sync_copy(buf, feats_hbm.at[pl.ds(w * ROWS_PER_W + j * EMB_CHUNK, EMB_CHUNK)])


@functools.partial(
    pl.kernel,
    out_type=jax.ShapeDtypeStruct((NC, N_AGG, D), jnp.float32),
    mesh=_mesh,
    scratch_types=[
        pltpu.VMEM((E_CHUNKS, E_CHUNK), jnp.int32),
        pltpu.VMEM((E_CHUNKS, E_CHUNK), jnp.int32),
        pltpu.VMEM((E_CHUNK, D), jnp.float32),
        pltpu.VMEM_SHARED((N_AGG, D), jnp.float32),
        pltpu.SemaphoreType.DMA,
    ],
)
def _edge_agg(x_hbm, src_hbm, dst_hbm, zeros_hbm, out_hbm,
              src_v, dst_v, buf, agg, sem):
    c = lax.axis_index("c")
    s = lax.axis_index("s")
    w = c * NS + s
    pltpu.sync_copy(zeros_hbm, agg.at[pl.ds(s * ROWS_PER_TILE, ROWS_PER_TILE)])
    pltpu.sync_copy(src_hbm.at[w], src_v)
    pltpu.sync_copy(dst_hbm.at[w], dst_v)
    plsc.subcore_barrier()

    def body(j, carry):
        pltpu.async_copy(x_hbm.at[src_v.at[j]], buf, sem).wait()
        pltpu.sync_copy(buf, agg.at[dst_v.at[j]], add=True)
        return carry

    lax.fori_loop(0, E_CHUNKS, body, 0)
    plsc.subcore_barrier()
    pltpu.sync_copy(agg.at[pl.ds(s * ROWS_PER_TILE, ROWS_PER_TILE)],
                    out_hbm.at[c, pl.ds(s * ROWS_PER_TILE, ROWS_PER_TILE)])


def _mm_body(p_ref, w_ref, b_ref, o_ref):
    a = p_ref[0] + p_ref[1]
    acc = jnp.dot(a, w_ref[...], preferred_element_type=jnp.float32)
    o_ref[...] = jnp.maximum(acc + b_ref[...], 0.0)


_MM_BLK = 1000


def _apply_linear_relu(partials, W, b):
    b2 = b.reshape(1, D)
    grid = N_NODES // _MM_BLK
    return pl.pallas_call(
        _mm_body,
        grid=(grid,),
        in_specs=[
            pl.BlockSpec((NC, _MM_BLK, D), lambda i: (0, i, 0)),
            pl.BlockSpec((D, D), lambda i: (0, 0)),
            pl.BlockSpec((1, D), lambda i: (0, 0)),
        ],
        out_specs=pl.BlockSpec((_MM_BLK, D), lambda i: (i, 0)),
        out_shape=jax.ShapeDtypeStruct((N_NODES, D), jnp.float32),
    )(partials, W, b2)


def kernel(cncpt_ids, edge_index, concept_table, W1, b1, W2, b2):
    pad = (jnp.arange(N_PAD - N_NODES, dtype=jnp.int32) * 419) % CONCEPT_NUM
    cidp = jnp.concatenate([cncpt_ids.astype(jnp.int32), pad]).reshape(
        NW, EMB_CHUNKS, EMB_CHUNK)
    src = edge_index[0].reshape(NW, E_CHUNKS, E_CHUNK)
    dst = edge_index[1].reshape(NW, E_CHUNKS, E_CHUNK)
    zeros = jnp.zeros((ROWS_PER_TILE, D), jnp.float32)

    feats = _embed_gather(concept_table, cidp)
    p1 = _edge_agg(feats, src, dst, zeros)
    h1 = _apply_linear_relu(p1, W1, b1)
    p2 = _edge_agg(h1, src, dst, zeros)
    h2 = _apply_linear_relu(p2, W2, b2)
    return h2


# speedup vs baseline: 12.6840x; 1.5045x over previous
import functools

import jax
import jax.numpy as jnp
from jax import lax
from jax.experimental import pallas as pl
from jax.experimental.pallas import tpu as pltpu
from jax.experimental.pallas import tpu_sc as plsc

N_NODES = 10000
N_EDGES = 320000
CONCEPT_NUM = 100000
D = 128

NC = 2
NS = 16
NW = NC * NS

N_PAD = 10240
EMB_CHUNKS = 5
EMB_CHUNK = 64
ROWS_PER_W = N_PAD // NW

E_STAGES = 4
E_CHUNKS_S = 27
E_CHUNK = 93
E_PAD = NW * E_STAGES * E_CHUNKS_S * E_CHUNK

N_AGG = 10112
ROWS_PER_TILE = N_AGG // NS

_mesh = plsc.VectorSubcoreMesh(core_axis_name="c", subcore_axis_name="s")


@functools.partial(
    pl.kernel,
    out_type=jax.ShapeDtypeStruct((N_PAD, D), jnp.float32),
    mesh=_mesh,
    scratch_types=[
        pltpu.VMEM((EMB_CHUNKS, EMB_CHUNK), jnp.int32),
        [pltpu.VMEM((EMB_CHUNK, D), jnp.float32) for _ in range(EMB_CHUNKS)],
        [pltpu.SemaphoreType.DMA for _ in range(EMB_CHUNKS)],
        pltpu.SemaphoreType.DMA,
    ],
)
def _embed_gather(table_hbm, cid_hbm, feats_hbm, idx_v, bufs, sems, wsem):
    c = lax.axis_index("c")
    s = lax.axis_index("s")
    w = c * NS + s
    pltpu.sync_copy(cid_hbm.at[w], idx_v)
    for j in range(EMB_CHUNKS):
        pltpu.async_copy(table_hbm.at[idx_v.at[j]], bufs[j], sems[j])
    for j in range(EMB_CHUNKS):
        out_slice = feats_hbm.at[pl.ds(w * ROWS_PER_W + j * EMB_CHUNK, EMB_CHUNK)]
        pltpu.make_async_copy(table_hbm.at[idx_v.at[j]], bufs[j], sems[j]).wait()
        pltpu.async_copy(bufs[j], out_slice, wsem)
    for j in range(EMB_CHUNKS):
        out_slice = feats_hbm.at[pl.ds(w * ROWS_PER_W + j * EMB_CHUNK, EMB_CHUNK)]
        pltpu.make_async_copy(bufs[j], out_slice, wsem).wait()


@functools.partial(
    pl.kernel,
    out_type=jax.ShapeDtypeStruct((NC, N_AGG, D), jnp.float32),
    mesh=_mesh,
    scratch_types=[
        pltpu.VMEM((E_CHUNKS_S, E_CHUNK), jnp.int32),
        pltpu.VMEM((E_CHUNKS_S, E_CHUNK), jnp.int32),
        pltpu.VMEM((E_CHUNK, D), jnp.float32),
        pltpu.VMEM((E_CHUNK, D), jnp.float32),
        pltpu.VMEM((E_CHUNK, D), jnp.float32),
        pltpu.VMEM_SHARED((N_AGG, D), jnp.float32),
        pltpu.SemaphoreType.DMA,
        pltpu.SemaphoreType.DMA,
        pltpu.SemaphoreType.DMA,
        pltpu.SemaphoreType.DMA,
        pltpu.SemaphoreType.DMA,
        pltpu.SemaphoreType.DMA,
    ],
)
def _edge_agg(x_hbm, src_hbm, dst_hbm, zeros_hbm, out_hbm,
              src_v, dst_v, buf0, buf1, buf2, agg, g0, g1, g2, s0, s1, s2):
    c = lax.axis_index("c")
    s = lax.axis_index("s")
    w = c * NS + s
    bufs = (buf0, buf1, buf2)
    gsem = (g0, g1, g2)
    ssem = (s0, s1, s2)
    pltpu.sync_copy(zeros_hbm, agg.at[pl.ds(s * ROWS_PER_TILE, ROWS_PER_TILE)])
    plsc.subcore_barrier()

    def g_start(j, b):
        pltpu.async_copy(x_hbm.at[src_v.at[j]], bufs[b], gsem[b])

    def g_wait(j, b):
        pltpu.make_async_copy(x_hbm.at[src_v.at[j]], bufs[b], gsem[b]).wait()

    def s_start(j, b):
        pltpu.async_copy(bufs[b], agg.at[dst_v.at[j]], ssem[b], add=True)

    def s_wait(j, b):
        pltpu.make_async_copy(bufs[b], agg.at[dst_v.at[j]], ssem[b]).wait()

    def body(i, carry):
        for t in range(3):
            j = 3 * i + 1 + t
            b = (1 + t) % 3
            g_wait(j, b)
            s_start(j, b)
            s_wait(j - 1, (b + 2) % 3)
            g_start(j + 2, (b + 2) % 3)
        return carry

    for h in range(E_STAGES):
        pltpu.sync_copy(src_hbm.at[w, h], src_v)
        pltpu.sync_copy(dst_hbm.at[w, h], dst_v)
        g_start(0, 0)
        g_start(1, 1)
        g_wait(0, 0)
        s_start(0, 0)
        g_start(2, 2)
        lax.fori_loop(0, (E_CHUNKS_S - 3) // 3, body, 0)
        ja = E_CHUNKS_S - 2
        jb = E_CHUNKS_S - 1
        g_wait(ja, ja % 3)
        s_start(ja, ja % 3)
        g_wait(jb, jb % 3)
        s_start(jb, jb % 3)
        s_wait(E_CHUNKS_S - 3, (E_CHUNKS_S - 3) % 3)
        s_wait(ja, ja % 3)
        s_wait(jb, jb % 3)
    plsc.subcore_barrier()
    pltpu.sync_copy(agg.at[pl.ds(s * ROWS_PER_TILE, ROWS_PER_TILE)],
                    out_hbm.at[c, pl.ds(s * ROWS_PER_TILE, ROWS_PER_TILE)])


def _mm_body(p_ref, w_ref, b_ref, o_ref):
    a = p_ref[0] + p_ref[1]
    acc = jnp.dot(a, w_ref[...], preferred_element_type=jnp.float32)
    o_ref[...] = jnp.maximum(acc + b_ref[...], 0.0)


_MM_BLK = 2000


def _apply_linear_relu(partials, W, b):
    b2 = b.reshape(1, D)
    grid = N_NODES // _MM_BLK
    return pl.pallas_call(
        _mm_body,
        grid=(grid,),
        in_specs=[
            pl.BlockSpec((NC, _MM_BLK, D), lambda i: (0, i, 0)),
            pl.BlockSpec((D, D), lambda i: (0, 0)),
            pl.BlockSpec((1, D), lambda i: (0, 0)),
        ],
        out_specs=pl.BlockSpec((_MM_BLK, D), lambda i: (i, 0)),
        out_shape=jax.ShapeDtypeStruct((N_NODES, D), jnp.float32),
    )(partials, W, b2)


def kernel(cncpt_ids, edge_index, concept_table, W1, b1, W2, b2):
    pad = (jnp.arange(N_PAD - N_NODES, dtype=jnp.int32) * 419) % CONCEPT_NUM
    cidp = jnp.concatenate([cncpt_ids.astype(jnp.int32), pad]).reshape(
        NW, EMB_CHUNKS, EMB_CHUNK)
    pad_n = E_PAD - N_EDGES
    src_pad = (jnp.arange(pad_n, dtype=jnp.int32) * 9973) % N_NODES
    dst_pad = N_NODES + (jnp.arange(pad_n, dtype=jnp.int32) % (N_AGG - N_NODES))
    src = jnp.concatenate([edge_index[0].astype(jnp.int32), src_pad]).reshape(
        NW, E_STAGES, E_CHUNKS_S, E_CHUNK)
    dst = jnp.concatenate([edge_index[1].astype(jnp.int32), dst_pad]).reshape(
        NW, E_STAGES, E_CHUNKS_S, E_CHUNK)
    zeros = jnp.zeros((ROWS_PER_TILE, D), jnp.float32)

    feats = _embed_gather(concept_table, cidp)
    p1 = _edge_agg(feats, src, dst, zeros)
    h1 = _apply_linear_relu(p1, W1, b1)
    p2 = _edge_agg(h1, src, dst, zeros)
    h2 = _apply_linear_relu(p2, W2, b2)
    return h2
